# baseline (speedup 1.0000x reference)
import functools

import jax
import jax.numpy as jnp
from jax import lax
from jax.experimental import pallas as pl
from jax.experimental.pallas import tpu as pltpu
from jax.experimental.pallas import tpu_sc as plsc

NUM_CORES = 2
NUM_SUBCORES = 16
NUM_WORKERS = NUM_CORES * NUM_SUBCORES
CHUNK = 128
EMBED_DIM = 128
ROW_PAD = 136


def _fuse_table_body(emb_ref, fr_ref, pr_ref, w1_ref, w2_ref, b_ref, out_ref):
    out_ref[...] = (
        emb_ref[...]
        + jnp.dot(fr_ref[...], w1_ref[...], preferred_element_type=jnp.float32)
        + jnp.dot(pr_ref[...], w2_ref[...], preferred_element_type=jnp.float32)
        + b_ref[...]
    )


def _fuse_table(emb, fr, pr, w1, w2, b):
    return pl.pallas_call(
        _fuse_table_body,
        out_shape=jax.ShapeDtypeStruct((ROW_PAD, EMBED_DIM), jnp.float32),
    )(emb, fr, pr, w1, w2, b)


@functools.partial(jax.jit, static_argnums=(2, 3))
def _gather_rows(table, idx2d, n_chunks_total, n_chunks_per_worker):
    mesh = plsc.VectorSubcoreMesh(core_axis_name="c", subcore_axis_name="s")

    n_groups = n_chunks_per_worker // 3
    n_groups -= n_groups % 6
    n_tail = n_chunks_per_worker - 3 * n_groups
    assert n_groups >= 6 and n_tail == 2

    @functools.partial(
        pl.kernel,
        mesh=mesh,
        out_type=jax.ShapeDtypeStruct((n_chunks_total * CHUNK, EMBED_DIM),
                                      jnp.float32),
        scratch_types=[
            pltpu.VMEM((ROW_PAD, EMBED_DIM), jnp.float32),
            pltpu.VMEM((n_chunks_per_worker, CHUNK), jnp.int32),
            [pltpu.VMEM((CHUNK, EMBED_DIM), jnp.float32)] * 3,
            [pltpu.VMEM((CHUNK, EMBED_DIM), jnp.float32)] * 2,
            [pltpu.SemaphoreType.DMA] * 3,
            [pltpu.SemaphoreType.DMA] * 3,
            [pltpu.SemaphoreType.DMA] * 2,
        ],
    )
    def gather(table_hbm, idx_hbm, out_hbm, table_v, idx_v, sbuf, cbuf,
               gsem, ssemS, ssemC):
        wid = lax.axis_index("s") * NUM_CORES + lax.axis_index("c")
        row0 = wid * n_chunks_per_worker * CHUNK
        base = wid * ROW_PAD + jnp.zeros((16,), jnp.int32)

        def out_slice(g):
            return out_hbm.at[pl.ds(row0 + g * CHUNK, CHUNK)]

        def gather_start(g, bi):
            pltpu.async_copy(table_hbm.at[idx_v.at[g]], sbuf[bi], gsem[bi])

        def gather_wait(g, bi):
            pltpu.make_async_copy(table_hbm.at[idx_v.at[g]], sbuf[bi],
                                  gsem[bi]).wait()

        def swait(g, bi):
            pltpu.make_async_copy(sbuf[bi], out_slice(g), ssemS[bi]).wait()

        def fill_chunk(cb, gc):
            @pl.loop(0, CHUNK // 16)
            def fill(j16):
                iv = idx_v[gc, pl.ds(j16 * 16, 16)] - base
                for t in range(16):
                    i = iv[t]
                    for k in range(EMBED_DIM // 16):
                        sl = pl.ds(k * 16, 16)
                        cbuf[cb][j16 * 16 + t, sl] = table_v[i, sl]

        pltpu.sync_copy(table_hbm.at[pl.ds(wid * ROW_PAD, ROW_PAD)], table_v)
        pltpu.sync_copy(idx_hbm.at[wid], idx_v)

        @pl.loop(0, n_chunks_per_worker)
        def adjust(g):
            for j in range(CHUNK // 16):
                sl = pl.ds(j * 16, 16)
                idx_v[g, sl] = idx_v[g, sl] + base

        gather_start(0, 0)
        gather_start(1, 1)

        @pl.loop(0, n_groups, step=6)
        def outer(t0):
            for u in range(6):
                t = t0 + u
                bA = (2 * u) % 3
                bB = (2 * u + 1) % 3
                bN = (2 * u + 2) % 3
                cb = u % 2
                gA = 3 * t
                gB = 3 * t + 1
                gC = 3 * t + 2

                @pl.when(3 * (t + 1) < n_chunks_per_worker)
                def _():
                    @pl.when(t >= 1)
                    def _():
                        swait(3 * (t - 1) + 1, bN)
                    gather_start(3 * (t + 1), bN)

                @pl.when(t >= 2)
                def _():
                    pltpu.make_async_copy(cbuf[cb], out_slice(gC),
                                          ssemC[cb]).wait()

                fill_chunk(cb, gC)
                gather_wait(gA, bA)
                pltpu.async_copy(sbuf[bA], out_slice(gA), ssemS[bA])
                gather_wait(gB, bB)
                pltpu.async_copy(sbuf[bB], out_slice(gB), ssemS[bB])
                pltpu.async_copy(cbuf[cb], out_slice(gC), ssemC[cb])

                @pl.when(3 * (t + 1) + 1 < n_chunks_per_worker)
                def _():
                    swait(gA, bA)
                    gather_start(3 * (t + 1) + 1, bA)

        for e in range(n_tail):
            g_tail = 3 * n_groups + e
            b_tail = (2 * n_groups + e) % 3
            gather_wait(g_tail, b_tail)
            pltpu.async_copy(sbuf[b_tail], out_slice(g_tail), ssemS[b_tail])

        for e in range(n_tail):
            g_tail = 3 * n_groups + e
            b_tail = (2 * n_groups + e) % 3
            swait(g_tail, b_tail)
        swait(3 * (n_groups - 1) + 1, (2 * (n_groups - 1) + 1) % 3)
        for cb in range(2):
            t_last = n_groups - 2 + cb
            pltpu.make_async_copy(cbuf[cb], out_slice(3 * t_last + 2),
                                  ssemC[cb]).wait()

    return gather(table, idx2d)


def kernel(instrument_indices, embedding_table, frequency_ranges,
           instrument_properties, W, b):
    batch, seq = instrument_indices.shape
    pad = ROW_PAD - embedding_table.shape[0]
    emb = jnp.pad(embedding_table, ((0, pad), (0, 0)))
    fr = jnp.pad(frequency_ranges, ((0, pad), (0, 0)))
    pr = jnp.pad(instrument_properties, ((0, pad), (0, 0)))
    fused = _fuse_table(emb, fr, pr, W[:fr.shape[1]], W[fr.shape[1]:],
                        b.reshape(1, EMBED_DIM))
    fused = jnp.tile(fused, (NUM_WORKERS, 1))

    total = batch * seq
    n_chunks_total = total // CHUNK
    n_chunks_per_worker = n_chunks_total // NUM_WORKERS
    idx2d = instrument_indices.reshape(
        NUM_WORKERS, n_chunks_per_worker, CHUNK).astype(jnp.int32)
    out = _gather_rows(fused, idx2d, n_chunks_total, n_chunks_per_worker)
    return out.reshape(batch, seq, EMBED_DIM)


# speedup vs baseline: 1.1766x; 1.1766x over previous
import functools

import jax
import jax.numpy as jnp
from jax import lax
from jax.experimental import pallas as pl
from jax.experimental.pallas import tpu as pltpu
from jax.experimental.pallas import tpu_sc as plsc

NUM_CORES = 2
NUM_SUBCORES = 16
NUM_WORKERS = NUM_CORES * NUM_SUBCORES
CHUNK = 128
EMBED_DIM = 128
ROW_PAD = 136


def _fuse_table_body(emb_ref, fr_ref, pr_ref, w1_ref, w2_ref, b_ref, out_ref):
    out_ref[...] = (
        emb_ref[...]
        + jnp.dot(fr_ref[...], w1_ref[...], preferred_element_type=jnp.float32)
        + jnp.dot(pr_ref[...], w2_ref[...], preferred_element_type=jnp.float32)
        + b_ref[...]
    )


def _fuse_table(emb, fr, pr, w1, w2, b):
    return pl.pallas_call(
        _fuse_table_body,
        out_shape=jax.ShapeDtypeStruct((ROW_PAD, EMBED_DIM), jnp.float32),
    )(emb, fr, pr, w1, w2, b)


@functools.partial(jax.jit, static_argnums=(2, 3))
def _gather_rows(table, idx2d, n_chunks_total, n_chunks_per_worker):
    mesh = plsc.VectorSubcoreMesh(core_axis_name="c", subcore_axis_name="s")

    nbuf = 5
    assert n_chunks_per_worker % nbuf == 0 and n_chunks_per_worker > nbuf

    @functools.partial(
        pl.kernel,
        mesh=mesh,
        out_type=jax.ShapeDtypeStruct((n_chunks_total * CHUNK, EMBED_DIM),
                                      jnp.float32),
        scratch_types=[
            pltpu.VMEM((n_chunks_per_worker, CHUNK), jnp.int32),
            [pltpu.VMEM((CHUNK, EMBED_DIM), jnp.float32)] * nbuf,
            [pltpu.SemaphoreType.DMA] * nbuf,
            [pltpu.SemaphoreType.DMA] * nbuf,
        ],
    )
    def gather(table_hbm, idx_hbm, out_hbm, idx_v, rows, gsem, ssem):
        wid = lax.axis_index("s") * NUM_CORES + lax.axis_index("c")
        row0 = wid * n_chunks_per_worker * CHUNK

        def gather_start(g, bi):
            pltpu.async_copy(table_hbm.at[idx_v.at[g]], rows[bi], gsem[bi])

        def gather_wait(g, bi):
            pltpu.make_async_copy(table_hbm.at[idx_v.at[g]], rows[bi],
                                  gsem[bi]).wait()

        def out_slice(g):
            return out_hbm.at[pl.ds(row0 + g * CHUNK, CHUNK)]

        pltpu.sync_copy(idx_hbm.at[wid], idx_v)

        base = (2 * wid * ROW_PAD
                + (lax.iota(jnp.int32, 16) % 2) * ROW_PAD)

        @pl.loop(0, n_chunks_per_worker)
        def adjust(g):
            for j in range(CHUNK // 16):
                sl = pl.ds(j * 16, 16)
                idx_v[g, sl] = idx_v[g, sl] + base

        for bi in range(nbuf):
            gather_start(bi, bi)

        @pl.loop(0, n_chunks_per_worker, step=nbuf)
        def outer(g0):
            for bi in range(nbuf):
                g = g0 + bi
                gather_wait(g, bi)
                pltpu.async_copy(rows[bi], out_slice(g), ssem[bi])

                @pl.when(g + nbuf < n_chunks_per_worker)
                def _():
                    pltpu.make_async_copy(rows[bi], out_slice(g),
                                          ssem[bi]).wait()
                    gather_start(g + nbuf, bi)

        for bi in range(nbuf):
            g_last = n_chunks_per_worker - nbuf + bi
            pltpu.make_async_copy(rows[bi], out_slice(g_last),
                                  ssem[bi]).wait()

    return gather(table, idx2d)


def kernel(instrument_indices, embedding_table, frequency_ranges,
           instrument_properties, W, b):
    batch, seq = instrument_indices.shape
    pad = ROW_PAD - embedding_table.shape[0]
    emb = jnp.pad(embedding_table, ((0, pad), (0, 0)))
    fr = jnp.pad(frequency_ranges, ((0, pad), (0, 0)))
    pr = jnp.pad(instrument_properties, ((0, pad), (0, 0)))
    fused = _fuse_table(emb, fr, pr, W[:fr.shape[1]], W[fr.shape[1]:],
                        b.reshape(1, EMBED_DIM))
    fused = jnp.tile(fused, (2 * NUM_WORKERS, 1))

    total = batch * seq
    n_chunks_total = total // CHUNK
    n_chunks_per_worker = n_chunks_total // NUM_WORKERS
    idx2d = instrument_indices.reshape(
        NUM_WORKERS, n_chunks_per_worker, CHUNK).astype(jnp.int32)
    out = _gather_rows(fused, idx2d, n_chunks_total, n_chunks_per_worker)
    return out.reshape(batch, seq, EMBED_DIM)


# speedup vs baseline: 1.2295x; 1.0450x over previous
import functools

import jax
import jax.numpy as jnp
from jax import lax
from jax.experimental import pallas as pl
from jax.experimental.pallas import tpu as pltpu
from jax.experimental.pallas import tpu_sc as plsc

NUM_CORES = 2
NUM_SUBCORES = 16
NUM_WORKERS = NUM_CORES * NUM_SUBCORES
CHUNK = 128
EMBED_DIM = 128
ROW_PAD = 136


def _fuse_table_body(emb_ref, fr_ref, pr_ref, w1_ref, w2_ref, b_ref, out_ref):
    out_ref[...] = (
        emb_ref[...]
        + jnp.dot(fr_ref[...], w1_ref[...], preferred_element_type=jnp.float32)
        + jnp.dot(pr_ref[...], w2_ref[...], preferred_element_type=jnp.float32)
        + b_ref[...]
    )


def _fuse_table(emb, fr, pr, w1, w2, b):
    return pl.pallas_call(
        _fuse_table_body,
        out_shape=jax.ShapeDtypeStruct((ROW_PAD, EMBED_DIM), jnp.float32),
    )(emb, fr, pr, w1, w2, b)


@functools.partial(jax.jit, static_argnums=(2, 3))
def _gather_rows(table, idx2d, n_chunks_total, n_chunks_per_worker):
    mesh = plsc.VectorSubcoreMesh(core_axis_name="c", subcore_axis_name="s")

    nbuf = 5
    assert n_chunks_per_worker % nbuf == 0 and n_chunks_per_worker > nbuf

    @functools.partial(
        pl.kernel,
        mesh=mesh,
        out_type=jax.ShapeDtypeStruct((n_chunks_total * CHUNK, EMBED_DIM),
                                      jnp.float32),
        scratch_types=[
            pltpu.VMEM((n_chunks_per_worker, CHUNK), jnp.int32),
            [pltpu.VMEM((CHUNK, EMBED_DIM), jnp.float32)] * nbuf,
            [pltpu.SemaphoreType.DMA] * nbuf,
            [pltpu.SemaphoreType.DMA] * nbuf,
        ],
    )
    def gather(table_hbm, idx_hbm, out_hbm, idx_v, rows, gsem, ssem):
        wid = lax.axis_index("s") * NUM_CORES + lax.axis_index("c")
        row0 = wid * n_chunks_per_worker * CHUNK

        def gather_start(g, bi):
            pltpu.async_copy(table_hbm.at[idx_v.at[g]], rows[bi], gsem[bi])

        def gather_wait(g, bi):
            pltpu.make_async_copy(table_hbm.at[idx_v.at[g]], rows[bi],
                                  gsem[bi]).wait()

        def out_slice(g):
            return out_hbm.at[pl.ds(row0 + g * CHUNK, CHUNK)]

        pltpu.sync_copy(idx_hbm.at[wid], idx_v)

        base = (4 * wid * ROW_PAD
                + (lax.iota(jnp.int32, 16) % 4) * ROW_PAD)

        @pl.loop(0, n_chunks_per_worker)
        def adjust(g):
            for j in range(CHUNK // 16):
                sl = pl.ds(j * 16, 16)
                idx_v[g, sl] = idx_v[g, sl] + base

        for bi in range(nbuf):
            gather_start(bi, bi)

        @pl.loop(0, n_chunks_per_worker, step=nbuf)
        def outer(g0):
            for bi in range(nbuf):
                g = g0 + bi
                gather_wait(g, bi)
                pltpu.async_copy(rows[bi], out_slice(g), ssem[bi])

                @pl.when(g + nbuf < n_chunks_per_worker)
                def _():
                    pltpu.make_async_copy(rows[bi], out_slice(g),
                                          ssem[bi]).wait()
                    gather_start(g + nbuf, bi)

        for bi in range(nbuf):
            g_last = n_chunks_per_worker - nbuf + bi
            pltpu.make_async_copy(rows[bi], out_slice(g_last),
                                  ssem[bi]).wait()

    return gather(table, idx2d)


def kernel(instrument_indices, embedding_table, frequency_ranges,
           instrument_properties, W, b):
    batch, seq = instrument_indices.shape
    pad = ROW_PAD - embedding_table.shape[0]
    emb = jnp.pad(embedding_table, ((0, pad), (0, 0)))
    fr = jnp.pad(frequency_ranges, ((0, pad), (0, 0)))
    pr = jnp.pad(instrument_properties, ((0, pad), (0, 0)))
    fused = _fuse_table(emb, fr, pr, W[:fr.shape[1]], W[fr.shape[1]:],
                        b.reshape(1, EMBED_DIM))
    fused = jnp.tile(fused, (4 * NUM_WORKERS, 1))

    total = batch * seq
    n_chunks_total = total // CHUNK
    n_chunks_per_worker = n_chunks_total // NUM_WORKERS
    idx2d = instrument_indices.reshape(
        NUM_WORKERS, n_chunks_per_worker, CHUNK).astype(jnp.int32)
    out = _gather_rows(fused, idx2d, n_chunks_total, n_chunks_per_worker)
    return out.reshape(batch, seq, EMBED_DIM)
